# baseline (speedup 1.0000x reference)
import jax
import jax.numpy as jnp
from jax import lax
from jax.experimental import pallas as pl
from jax.experimental.pallas import tpu as pltpu
from jax.experimental.pallas import tpu_sc as plsc

_NUM_STEPS = 1000
_BATCH = 16384

_NC = 2
_NS = 16
_L = 16
_NW = _NC * _NS
_BPW = _BATCH // _NW


def _body(t_hbm, beta_hbm, alpha_hbm, beta_out, alpha_out,
          idx_v, beta_tab, alpha_tab, bout_v, aout_v, in_sem, out_sem):
    wid = lax.axis_index("s") * _NC + lax.axis_index("c")
    base = wid * _BPW
    cp_b = pltpu.async_copy(beta_hbm, beta_tab, in_sem)
    cp_a = pltpu.async_copy(alpha_hbm, alpha_tab, in_sem)
    cp_t = pltpu.async_copy(t_hbm.at[pl.ds(base, _BPW)], idx_v, in_sem)
    cp_b.wait()
    cp_a.wait()
    cp_t.wait()
    half = _BPW // 2
    outs = []
    for h in range(2):
        lo = h * half

        @plsc.parallel_loop(lo, lo + half, step=_L, unroll=4)
        def _gather(i):
            idx = idx_v[pl.ds(i, _L)]
            bout_v[pl.ds(i, _L)] = plsc.load_gather(beta_tab, [idx])
            aout_v[pl.ds(i, _L)] = plsc.load_gather(alpha_tab, [idx])

        outs.append(pltpu.async_copy(
            bout_v.at[pl.ds(lo, half)],
            beta_out.at[pl.ds(base + lo, half)], out_sem))
        outs.append(pltpu.async_copy(
            aout_v.at[pl.ds(lo, half)],
            alpha_out.at[pl.ds(base + lo, half)], out_sem))
    for cp in outs:
        cp.wait()


def kernel(t, beta, alpha):
    mesh = plsc.VectorSubcoreMesh(core_axis_name="c", subcore_axis_name="s")
    f = pl.kernel(
        _body,
        mesh=mesh,
        compiler_params=pltpu.CompilerParams(needs_layout_passes=False,
                                             skip_device_barrier=True),
        out_type=(
            jax.ShapeDtypeStruct((_BATCH,), jnp.float32),
            jax.ShapeDtypeStruct((_BATCH,), jnp.float32),
        ),
        scratch_types=[
            pltpu.VMEM((_BPW,), jnp.int32),
            pltpu.VMEM((_NUM_STEPS,), jnp.float32),
            pltpu.VMEM((_NUM_STEPS,), jnp.float32),
            pltpu.VMEM((_BPW,), jnp.float32),
            pltpu.VMEM((_BPW,), jnp.float32),
            pltpu.SemaphoreType.DMA,
            pltpu.SemaphoreType.DMA,
        ],
    )
    beta_t, alpha_t = f(t.astype(jnp.int32), beta, alpha)
    return (beta_t, alpha_t)


# speedup vs baseline: 1.1077x; 1.1077x over previous
import jax
import jax.numpy as jnp
from jax import lax
from jax.experimental import pallas as pl
from jax.experimental.pallas import tpu as pltpu
from jax.experimental.pallas import tpu_sc as plsc

_NUM_STEPS = 1000
_BATCH = 16384

_NC = 1
_NS = 16
_L = 16
_NW = _NC * _NS
_BPW = _BATCH // _NW


def _body(t_hbm, beta_hbm, alpha_hbm, beta_out, alpha_out,
          idx_v, beta_tab, alpha_tab, bout_v, aout_v, in_sem, out_sem):
    wid = lax.axis_index("s") * _NC + lax.axis_index("c")
    base = wid * _BPW
    cp_b = pltpu.async_copy(beta_hbm, beta_tab, in_sem)
    cp_a = pltpu.async_copy(alpha_hbm, alpha_tab, in_sem)
    cp_t = pltpu.async_copy(t_hbm.at[pl.ds(base, _BPW)], idx_v, in_sem)
    cp_b.wait()
    cp_a.wait()
    cp_t.wait()
    half = _BPW // 2
    outs = []
    for h in range(2):
        lo = h * half

        @plsc.parallel_loop(lo, lo + half, step=_L, unroll=4)
        def _gather(i):
            idx = idx_v[pl.ds(i, _L)]
            bout_v[pl.ds(i, _L)] = plsc.load_gather(beta_tab, [idx])
            aout_v[pl.ds(i, _L)] = plsc.load_gather(alpha_tab, [idx])

        outs.append(pltpu.async_copy(
            bout_v.at[pl.ds(lo, half)],
            beta_out.at[pl.ds(base + lo, half)], out_sem))
        outs.append(pltpu.async_copy(
            aout_v.at[pl.ds(lo, half)],
            alpha_out.at[pl.ds(base + lo, half)], out_sem))
    for cp in outs:
        cp.wait()


def kernel(t, beta, alpha):
    mesh = plsc.VectorSubcoreMesh(core_axis_name="c", subcore_axis_name="s",
                                  num_cores=_NC)
    f = pl.kernel(
        _body,
        mesh=mesh,
        compiler_params=pltpu.CompilerParams(needs_layout_passes=False),
        out_type=(
            jax.ShapeDtypeStruct((_BATCH,), jnp.float32),
            jax.ShapeDtypeStruct((_BATCH,), jnp.float32),
        ),
        scratch_types=[
            pltpu.VMEM((_BPW,), jnp.int32),
            pltpu.VMEM((_NUM_STEPS,), jnp.float32),
            pltpu.VMEM((_NUM_STEPS,), jnp.float32),
            pltpu.VMEM((_BPW,), jnp.float32),
            pltpu.VMEM((_BPW,), jnp.float32),
            pltpu.SemaphoreType.DMA,
            pltpu.SemaphoreType.DMA,
        ],
    )
    beta_t, alpha_t = f(t.astype(jnp.int32), beta, alpha)
    return (beta_t, alpha_t)


# speedup vs baseline: 1.1202x; 1.0113x over previous
import jax
import jax.numpy as jnp
from jax import lax
from jax.experimental import pallas as pl
from jax.experimental.pallas import tpu as pltpu
from jax.experimental.pallas import tpu_sc as plsc

_NUM_STEPS = 1000
_BATCH = 16384

_NC = 1
_NS = 16
_L = 16
_NW = _NC * _NS
_BPW = _BATCH // _NW


def _body(t_hbm, beta_hbm, alpha_hbm, beta_out, alpha_out,
          idx_v, beta_tab, alpha_tab, bout_v, aout_v, in_sem, out_sem):
    wid = lax.axis_index("s") * _NC + lax.axis_index("c")
    base = wid * _BPW
    cp_t = pltpu.async_copy(t_hbm.at[pl.ds(base, _BPW)], idx_v, in_sem)
    cp_b = pltpu.async_copy(beta_hbm, beta_tab, in_sem)
    cp_a = pltpu.async_copy(alpha_hbm, alpha_tab, in_sem)
    cp_t.wait()
    cp_b.wait()

    @plsc.parallel_loop(0, _BPW, step=_L, unroll=4)
    def _gather_b(i):
        idx = idx_v[pl.ds(i, _L)]
        bout_v[pl.ds(i, _L)] = plsc.load_gather(beta_tab, [idx])

    cp_ob = pltpu.async_copy(bout_v, beta_out.at[pl.ds(base, _BPW)], out_sem)
    cp_a.wait()

    @plsc.parallel_loop(0, _BPW, step=_L, unroll=4)
    def _gather_a(i):
        idx = idx_v[pl.ds(i, _L)]
        aout_v[pl.ds(i, _L)] = plsc.load_gather(alpha_tab, [idx])

    cp_oa = pltpu.async_copy(aout_v, alpha_out.at[pl.ds(base, _BPW)], out_sem)
    cp_ob.wait()
    cp_oa.wait()


def kernel(t, beta, alpha):
    mesh = plsc.VectorSubcoreMesh(core_axis_name="c", subcore_axis_name="s",
                                  num_cores=_NC)
    f = pl.kernel(
        _body,
        mesh=mesh,
        compiler_params=pltpu.CompilerParams(needs_layout_passes=False),
        out_type=(
            jax.ShapeDtypeStruct((_BATCH,), jnp.float32),
            jax.ShapeDtypeStruct((_BATCH,), jnp.float32),
        ),
        scratch_types=[
            pltpu.VMEM((_BPW,), jnp.int32),
            pltpu.VMEM((_NUM_STEPS,), jnp.float32),
            pltpu.VMEM((_NUM_STEPS,), jnp.float32),
            pltpu.VMEM((_BPW,), jnp.float32),
            pltpu.VMEM((_BPW,), jnp.float32),
            pltpu.SemaphoreType.DMA,
            pltpu.SemaphoreType.DMA,
        ],
    )
    beta_t, alpha_t = f(t.astype(jnp.int32), beta, alpha)
    return (beta_t, alpha_t)


# speedup vs baseline: 1.1226x; 1.0022x over previous
import jax
import jax.numpy as jnp
from jax import lax
from jax.experimental import pallas as pl
from jax.experimental.pallas import tpu as pltpu
from jax.experimental.pallas import tpu_sc as plsc

_NUM_STEPS = 1000
_BATCH = 16384

_NC = 1
_NS = 16
_L = 16
_NW = _NC * _NS
_BPW = _BATCH // _NW


def _body(t_hbm, beta_hbm, alpha_hbm, beta_out, alpha_out,
          idx_v, beta_tab, alpha_tab, bout_v, aout_v, in_sem, out_sem):
    wid = lax.axis_index("s") * _NC + lax.axis_index("c")
    base = wid * _BPW
    cp_t = pltpu.async_copy(t_hbm.at[pl.ds(base, _BPW)], idx_v, in_sem)
    cp_b = pltpu.async_copy(beta_hbm, beta_tab, in_sem)
    cp_a = pltpu.async_copy(alpha_hbm, alpha_tab, in_sem)
    cp_t.wait()
    cp_b.wait()

    @plsc.parallel_loop(0, _BPW, step=_L, unroll=8)
    def _gather_b(i):
        idx = idx_v[pl.ds(i, _L)]
        bout_v[pl.ds(i, _L)] = plsc.load_gather(beta_tab, [idx])

    cp_ob = pltpu.async_copy(bout_v, beta_out.at[pl.ds(base, _BPW)], out_sem)
    cp_a.wait()

    @plsc.parallel_loop(0, _BPW, step=_L, unroll=8)
    def _gather_a(i):
        idx = idx_v[pl.ds(i, _L)]
        aout_v[pl.ds(i, _L)] = plsc.load_gather(alpha_tab, [idx])

    cp_oa = pltpu.async_copy(aout_v, alpha_out.at[pl.ds(base, _BPW)], out_sem)
    cp_ob.wait()
    cp_oa.wait()


def kernel(t, beta, alpha):
    mesh = plsc.VectorSubcoreMesh(core_axis_name="c", subcore_axis_name="s",
                                  num_cores=_NC)
    f = pl.kernel(
        _body,
        mesh=mesh,
        compiler_params=pltpu.CompilerParams(needs_layout_passes=False),
        out_type=(
            jax.ShapeDtypeStruct((_BATCH,), jnp.float32),
            jax.ShapeDtypeStruct((_BATCH,), jnp.float32),
        ),
        scratch_types=[
            pltpu.VMEM((_BPW,), jnp.int32),
            pltpu.VMEM((_NUM_STEPS,), jnp.float32),
            pltpu.VMEM((_NUM_STEPS,), jnp.float32),
            pltpu.VMEM((_BPW,), jnp.float32),
            pltpu.VMEM((_BPW,), jnp.float32),
            pltpu.SemaphoreType.DMA,
            pltpu.SemaphoreType.DMA,
        ],
    )
    beta_t, alpha_t = f(t.astype(jnp.int32), beta, alpha)
    return (beta_t, alpha_t)


# speedup vs baseline: 1.1502x; 1.0246x over previous
import jax
import jax.numpy as jnp
from jax import lax
from jax.experimental import pallas as pl
from jax.experimental.pallas import tpu as pltpu
from jax.experimental.pallas import tpu_sc as plsc

_NUM_STEPS = 1000
_BATCH = 16384

_NC = 1
_NS = 16
_L = 16
_NW = _NC * _NS
_BPW = _BATCH // _NW


def _body(t_hbm, beta_hbm, alpha_hbm, beta_out, alpha_out,
          idx_v, beta_tab, bout_v, in_sem, out_sem):
    wid = lax.axis_index("s") * _NC + lax.axis_index("c")
    seg = wid // 2
    base = seg * (2 * _BPW)
    cp_t = pltpu.async_copy(t_hbm.at[pl.ds(base, 2 * _BPW)], idx_v, in_sem)

    @pl.when(wid % 2 == 0)
    def _beta_tile():
        cp_tab = pltpu.async_copy(beta_hbm, beta_tab, in_sem)
        cp_t.wait()
        cp_tab.wait()

        @plsc.parallel_loop(0, 2 * _BPW, step=_L, unroll=8)
        def _gather(i):
            idx = idx_v[pl.ds(i, _L)]
            bout_v[pl.ds(i, _L)] = plsc.load_gather(beta_tab, [idx])

        pltpu.async_copy(bout_v, beta_out.at[pl.ds(base, 2 * _BPW)],
                         out_sem).wait()

    @pl.when(wid % 2 == 1)
    def _alpha_tile():
        cp_tab = pltpu.async_copy(alpha_hbm, beta_tab, in_sem)
        cp_t.wait()
        cp_tab.wait()

        @plsc.parallel_loop(0, 2 * _BPW, step=_L, unroll=8)
        def _gather(i):
            idx = idx_v[pl.ds(i, _L)]
            bout_v[pl.ds(i, _L)] = plsc.load_gather(beta_tab, [idx])

        pltpu.async_copy(bout_v, alpha_out.at[pl.ds(base, 2 * _BPW)],
                         out_sem).wait()


def kernel(t, beta, alpha):
    mesh = plsc.VectorSubcoreMesh(core_axis_name="c", subcore_axis_name="s",
                                  num_cores=_NC)
    f = pl.kernel(
        _body,
        mesh=mesh,
        compiler_params=pltpu.CompilerParams(needs_layout_passes=False),
        out_type=(
            jax.ShapeDtypeStruct((_BATCH,), jnp.float32),
            jax.ShapeDtypeStruct((_BATCH,), jnp.float32),
        ),
        scratch_types=[
            pltpu.VMEM((2 * _BPW,), jnp.int32),
            pltpu.VMEM((_NUM_STEPS,), jnp.float32),
            pltpu.VMEM((2 * _BPW,), jnp.float32),
            pltpu.SemaphoreType.DMA,
            pltpu.SemaphoreType.DMA,
        ],
    )
    beta_t, alpha_t = f(t.astype(jnp.int32), beta, alpha)
    return (beta_t, alpha_t)
